# baseline (speedup 1.0000x reference)
import jax
import jax.numpy as jnp
import numpy as np
from jax.experimental import pallas as pl
from jax.experimental.pallas import tpu as pltpu

_MIN_CONF = 0.7
_NMS_THR = 0.3
_MAX_INST = 100
_ACC_CAP = 160
_NEG = float("-inf")


def _prep_kernel(n_ref, probs_ref, deltas_ref, rois_ref, win_ref,
                 y1_ref, x1_ref, y2_ref, x2_ref, msc_ref, cid_ref):
    probs = probs_ref[...]
    num_c, nb, nl = probs.shape
    maxv = jnp.max(probs, axis=0)
    ci3 = jax.lax.broadcasted_iota(jnp.int32, (num_c, nb, nl), 0)
    cid = jnp.min(jnp.where(probs == maxv[None], ci3, num_c), axis=0)

    dl = deltas_ref[...]
    ci4 = jax.lax.broadcasted_iota(jnp.int32, (nb, num_c, 4, nl), 1)
    dsel = jnp.sum(jnp.where(ci4 == cid[:, None, None, :], dl, 0.0),
                   axis=1)
    dy = dsel[:, 0, :] * 0.1
    dx = dsel[:, 1, :] * 0.1
    dh = dsel[:, 2, :] * 0.2
    dw = dsel[:, 3, :] * 0.2

    r = rois_ref[...]
    ry1, rx1, ry2, rx2 = r[:, 0, :], r[:, 1, :], r[:, 2, :], r[:, 3, :]
    height = ry2 - ry1
    width = rx2 - rx1
    cy = ry1 + 0.5 * height + dy * height
    cx = rx1 + 0.5 * width + dx * width
    h = height * jnp.exp(dh)
    w = width * jnp.exp(dw)
    y1 = cy - 0.5 * h
    x1 = cx - 0.5 * w
    y2 = y1 + h
    x2 = x1 + w
    win = win_ref[...]
    wy1, wx1 = win[:, 0:1], win[:, 1:2]
    wy2, wx2 = win[:, 2:3], win[:, 3:4]
    y1_ref[...] = jnp.clip(y1, wy1, wy2)
    x1_ref[...] = jnp.clip(x1, wx1, wx2)
    y2_ref[...] = jnp.clip(y2, wy1, wy2)
    x2_ref[...] = jnp.clip(x2, wx1, wx2)

    n = n_ref[0, 0]
    t = pl.program_id(0)
    lane = jax.lax.broadcasted_iota(jnp.int32, (nb, nl), 1) + t * nl
    elig = (cid > 0) & (maxv >= _MIN_CONF) & (lane < n)
    msc_ref[...] = jnp.where(elig, maxv, _NEG)
    cid_ref[...] = cid


_CHUNK = 256


def _select_kernel(y1_ref, x1_ref, y2_ref, x2_ref, msc_ref, cid_ref, out_ref,
                   msc_s):
    nb, n = y1_ref.shape
    nblk = n // _CHUNK

    iota_chunk = jax.lax.broadcasted_iota(jnp.int32, (1, _CHUNK), 1)
    iota_blk = jax.lax.broadcasted_iota(jnp.int32, (1, nblk), 1)
    iota_cap = jax.lax.broadcasted_iota(jnp.int32, (1, _ACC_CAP), 1)
    iota_cls = jax.lax.broadcasted_iota(jnp.int32, (1, 128), 1)
    neg = jnp.float32(_NEG)
    msc_s[...] = msc_ref[...]

    def binit(j, bms):
        base = pl.multiple_of(j * _CHUNK, _CHUNK)
        out = []
        for b in range(nb):
            mj = jnp.max(msc_s[b:b + 1, pl.ds(base, _CHUNK)])
            out.append(jnp.where(iota_blk == j, mj, bms[b]))
        return tuple(out)
    bmax0 = jax.lax.fori_loop(
        0, nblk, binit,
        tuple(jnp.full((1, nblk), _NEG, jnp.float32) for _ in range(nb)))

    def _flag(m, kth, nacc):
        return ((m > neg) & ((nacc < _MAX_INST)
                             | ((m >= kth) & (nacc < _ACC_CAP)))
                ).astype(jnp.int32)

    def _mk_state(bm):
        m0 = jnp.max(bm, keepdims=True)
        kth0 = jnp.full((1, 1), _NEG, jnp.float32)
        nacc0 = jnp.zeros((1, 1), jnp.int32)
        return (
            bm,
            jnp.zeros((1, 128), jnp.int32),
            jnp.full((1, _ACC_CAP), _NEG, jnp.float32),
            jnp.zeros((1, _ACC_CAP), jnp.float32),
            jnp.zeros((1, _ACC_CAP), jnp.float32),
            jnp.zeros((1, _ACC_CAP), jnp.float32),
            jnp.zeros((1, _ACC_CAP), jnp.float32),
            jnp.zeros((1, _ACC_CAP), jnp.float32),
            jnp.full((1, _ACC_CAP), -1, jnp.int32),
            jnp.zeros((1, _ACC_CAP), jnp.int32),
            nacc0,
            m0,
            kth0,
            _flag(m0, kth0, nacc0),
        )

    def _step(b, st):
        (bmax, counts, aS, aY1, aX1, aY2, aX2, aA, aC, aR, nacc,
         m, kth, flag) = st
        live = flag != 0
        j = jnp.min(jnp.where(bmax == m, iota_blk, nblk - 1))
        base = pl.multiple_of(j * _CHUNK, _CHUNK)
        mblk = msc_s[b:b + 1, pl.ds(base, _CHUNK)]
        lane = jnp.min(jnp.where(mblk == m, iota_chunk, _CHUNK - 1),
                       keepdims=True)
        pick = iota_chunk == lane
        ci_ = jnp.sum(jnp.where(
            pick, cid_ref[b:b + 1, pl.ds(base, _CHUNK)], 0), keepdims=True)
        by1 = jnp.sum(jnp.where(
            pick, y1_ref[b:b + 1, pl.ds(base, _CHUNK)], 0.0), keepdims=True)
        bx1 = jnp.sum(jnp.where(
            pick, x1_ref[b:b + 1, pl.ds(base, _CHUNK)], 0.0), keepdims=True)
        by2 = jnp.sum(jnp.where(
            pick, y2_ref[b:b + 1, pl.ds(base, _CHUNK)], 0.0), keepdims=True)
        bx2 = jnp.sum(jnp.where(
            pick, x2_ref[b:b + 1, pl.ds(base, _CHUNK)], 0.0), keepdims=True)
        ba = (by2 - by1) * (bx2 - bx1)

        yy1 = jnp.maximum(by1, aY1)
        xx1 = jnp.maximum(bx1, aX1)
        yy2 = jnp.minimum(by2, aY2)
        xx2 = jnp.minimum(bx2, aX2)
        inter = jnp.maximum(0.0, yy2 - yy1) * jnp.maximum(0.0, xx2 - xx1)
        union = ba + aA - inter
        iou = jnp.where(union > 0, inter / jnp.maximum(union, 1e-12), 0.0)
        samecls = (aC == ci_) & (iota_cap < nacc)
        suppressed = jnp.any(samecls & (iou > _NMS_THR), keepdims=True)
        cnt_c = jnp.sum(jnp.where(iota_cls == ci_, counts, 0), keepdims=True)
        accept = live & jnp.logical_not(suppressed) & (cnt_c < _MAX_INST)

        slotmask = (iota_cap == nacc) & accept
        aS = jnp.where(slotmask, m, aS)
        aY1 = jnp.where(slotmask, by1, aY1)
        aX1 = jnp.where(slotmask, bx1, aX1)
        aY2 = jnp.where(slotmask, by2, aY2)
        aX2 = jnp.where(slotmask, bx2, aX2)
        aA = jnp.where(slotmask, ba, aA)
        aC = jnp.where(slotmask, ci_, aC)
        aR = jnp.where(slotmask, cnt_c, aR)
        counts = jnp.where((iota_cls == ci_) & accept, counts + 1, counts)
        kth = jnp.where(accept & (nacc == _MAX_INST - 1), m, kth)
        nacc = nacc + accept.astype(jnp.int32)
        mblk_new = jnp.where(pick & live, neg, mblk)
        msc_s[b:b + 1, pl.ds(base, _CHUNK)] = mblk_new
        bmax = jnp.where((iota_blk == j) & live,
                         jnp.max(mblk_new, keepdims=True), bmax)
        m = jnp.max(bmax, keepdims=True)
        return (bmax, counts, aS, aY1, aX1, aY2, aX2, aA, aC, aR, nacc,
                m, kth, _flag(m, kth, nacc))

    def cond(sts):
        total = sts[0][13]
        for b in range(1, nb):
            total = total + sts[b][13]
        return jnp.sum(total) > 0

    def body(sts):
        sts = tuple(_step(b, sts[b]) for b in range(nb))
        return tuple(_step(b, sts[b]) for b in range(nb))

    sts = jax.lax.while_loop(cond, body, tuple(_mk_state(bm) for bm in bmax0))

    rowi = jax.lax.broadcasted_iota(jnp.int32, (_MAX_INST, 6), 0)
    colj = jax.lax.broadcasted_iota(jnp.int32, (_MAX_INST, 6), 1)
    bigi = jnp.int32(1 << 30)
    grs = [sts[b][8] * _MAX_INST + sts[b][9] for b in range(nb)]

    def _order(b, t, st2):
        det, avail_i = st2
        aS, aY1, aX1, aY2, aX2, aA, aC, aR = sts[b][2:10]
        gr = grs[b]
        avail = avail_i != 0
        mm = jnp.max(jnp.where(avail, aS, neg), keepdims=True)
        cand = avail & (aS == mm)
        g = jnp.min(jnp.where(cand, gr, bigi), keepdims=True)
        slot = jnp.min(jnp.where(cand & (gr == g), iota_cap, _ACC_CAP),
                       keepdims=True)
        pickc = iota_cap == slot
        vy1 = jnp.sum(jnp.where(pickc, aY1, 0.0), keepdims=True)
        vx1 = jnp.sum(jnp.where(pickc, aX1, 0.0), keepdims=True)
        vy2 = jnp.sum(jnp.where(pickc, aY2, 0.0), keepdims=True)
        vx2 = jnp.sum(jnp.where(pickc, aX2, 0.0), keepdims=True)
        vc = jnp.sum(jnp.where(pickc, aC, 0), keepdims=True
                     ).astype(jnp.float32)
        vs = jnp.sum(jnp.where(pickc, aS, 0.0), keepdims=True)
        ok = mm > neg
        rowm = (rowi == t) & ok
        rowvals = jnp.where(colj == 0, vy1,
                   jnp.where(colj == 1, vx1,
                    jnp.where(colj == 2, vy2,
                     jnp.where(colj == 3, vx2,
                      jnp.where(colj == 4, vc, vs)))))
        det = jnp.where(rowm, rowvals, det)
        avail_i = jnp.where(iota_cap != slot, avail_i, 0)
        return det, avail_i

    def obody(t, st_all):
        return tuple(_order(b, t, st_all[b]) for b in range(nb))

    det0s = tuple(
        (jnp.zeros((_MAX_INST, 6), jnp.float32),
         (iota_cap < sts[b][10]).astype(jnp.int32))
        for b in range(nb))
    st_all = jax.lax.fori_loop(0, _MAX_INST, obody, det0s)
    for b in range(nb):
        out_ref[b] = st_all[b][0]


def kernel(rois, mrcnn_class, mrcnn_bbox, image_meta):
    B, N, C = mrcnn_class.shape
    shift = jnp.asarray(np.array([0.0, 0.0, 1.0, 1.0], dtype=np.float32))
    image_shape = image_meta[0, 4:7]
    scale = jnp.concatenate([image_shape[:2], image_shape[:2]]) - 1.0
    windows = (image_meta[:, 7:11] - shift) / scale

    probs_t = jnp.transpose(mrcnn_class, (2, 0, 1))
    bbox_t = jnp.transpose(mrcnn_bbox, (0, 2, 3, 1))
    rois_t = jnp.transpose(rois, (0, 2, 1))

    NL = 2048
    T = -(-N // NL)
    n_lanes = T * NL
    n_arr = jnp.full((1, 1), N, jnp.int32)

    outs = pl.pallas_call(
        _prep_kernel,
        grid=(T,),
        in_specs=[
            pl.BlockSpec((1, 1), lambda t: (0, 0)),
            pl.BlockSpec((C, B, NL), lambda t: (0, 0, t)),
            pl.BlockSpec((B, C, 4, NL), lambda t: (0, 0, 0, t)),
            pl.BlockSpec((B, 4, NL), lambda t: (0, 0, t)),
            pl.BlockSpec((B, 4), lambda t: (0, 0)),
        ],
        out_specs=[
            pl.BlockSpec((B, NL), lambda t: (0, t)),
            pl.BlockSpec((B, NL), lambda t: (0, t)),
            pl.BlockSpec((B, NL), lambda t: (0, t)),
            pl.BlockSpec((B, NL), lambda t: (0, t)),
            pl.BlockSpec((B, NL), lambda t: (0, t)),
            pl.BlockSpec((B, NL), lambda t: (0, t)),
        ],
        out_shape=[
            jax.ShapeDtypeStruct((B, n_lanes), jnp.float32),
            jax.ShapeDtypeStruct((B, n_lanes), jnp.float32),
            jax.ShapeDtypeStruct((B, n_lanes), jnp.float32),
            jax.ShapeDtypeStruct((B, n_lanes), jnp.float32),
            jax.ShapeDtypeStruct((B, n_lanes), jnp.float32),
            jax.ShapeDtypeStruct((B, n_lanes), jnp.int32),
        ],
    )(n_arr, probs_t, bbox_t, rois_t, windows)
    y1, x1, y2, x2, msc, cid = outs

    det = pl.pallas_call(
        _select_kernel,
        in_specs=[
            pl.BlockSpec((B, n_lanes), lambda: (0, 0)),
            pl.BlockSpec((B, n_lanes), lambda: (0, 0)),
            pl.BlockSpec((B, n_lanes), lambda: (0, 0)),
            pl.BlockSpec((B, n_lanes), lambda: (0, 0)),
            pl.BlockSpec((B, n_lanes), lambda: (0, 0)),
            pl.BlockSpec((B, n_lanes), lambda: (0, 0)),
        ],
        out_specs=pl.BlockSpec((B, _MAX_INST, 6), lambda: (0, 0, 0)),
        out_shape=jax.ShapeDtypeStruct((B, _MAX_INST, 6), jnp.float32),
        scratch_shapes=[pltpu.VMEM((B, n_lanes), jnp.float32)],
    )(y1, x1, y2, x2, msc, cid)
    return det


# speedup vs baseline: 1.0122x; 1.0122x over previous
import jax
import jax.numpy as jnp
import numpy as np
from jax.experimental import pallas as pl
from jax.experimental.pallas import tpu as pltpu

_MIN_CONF = 0.7
_NMS_THR = 0.3
_MAX_INST = 100
_ACC_CAP = 160
_NEG = float("-inf")


def _prep_kernel(n_ref, probs_ref, deltas_ref, rois_ref, win_ref,
                 y1_ref, x1_ref, y2_ref, x2_ref, msc_ref, cid_ref):
    probs = probs_ref[...]
    num_c, nb, nl = probs.shape
    maxv = jnp.max(probs, axis=0)
    ci3 = jax.lax.broadcasted_iota(jnp.int32, (num_c, nb, nl), 0)
    cid = jnp.min(jnp.where(probs == maxv[None], ci3, num_c), axis=0)

    dl = deltas_ref[...]
    ci4 = jax.lax.broadcasted_iota(jnp.int32, (nb, num_c, 4, nl), 1)
    dsel = jnp.sum(jnp.where(ci4 == cid[:, None, None, :], dl, 0.0),
                   axis=1)
    dy = dsel[:, 0, :] * 0.1
    dx = dsel[:, 1, :] * 0.1
    dh = dsel[:, 2, :] * 0.2
    dw = dsel[:, 3, :] * 0.2

    r = rois_ref[...]
    ry1, rx1, ry2, rx2 = r[:, 0, :], r[:, 1, :], r[:, 2, :], r[:, 3, :]
    height = ry2 - ry1
    width = rx2 - rx1
    cy = ry1 + 0.5 * height + dy * height
    cx = rx1 + 0.5 * width + dx * width
    h = height * jnp.exp(dh)
    w = width * jnp.exp(dw)
    y1 = cy - 0.5 * h
    x1 = cx - 0.5 * w
    y2 = y1 + h
    x2 = x1 + w
    win = win_ref[...]
    wy1, wx1 = win[:, 0:1], win[:, 1:2]
    wy2, wx2 = win[:, 2:3], win[:, 3:4]
    y1_ref[...] = jnp.clip(y1, wy1, wy2)
    x1_ref[...] = jnp.clip(x1, wx1, wx2)
    y2_ref[...] = jnp.clip(y2, wy1, wy2)
    x2_ref[...] = jnp.clip(x2, wx1, wx2)

    n = n_ref[0, 0]
    t = pl.program_id(0)
    lane = jax.lax.broadcasted_iota(jnp.int32, (nb, nl), 1) + t * nl
    elig = (cid > 0) & (maxv >= _MIN_CONF) & (lane < n)
    msc_ref[...] = jnp.where(elig, maxv, _NEG)
    cid_ref[...] = cid


_CHUNK = 256


def _select_kernel(y1_ref, x1_ref, y2_ref, x2_ref, msc_ref, cid_ref, out_ref,
                   msc_s):
    nb, n = y1_ref.shape
    nblk = n // _CHUNK

    iota_chunk = jax.lax.broadcasted_iota(jnp.int32, (1, _CHUNK), 1)
    iota_blk = jax.lax.broadcasted_iota(jnp.int32, (1, nblk), 1)
    iota_cap = jax.lax.broadcasted_iota(jnp.int32, (1, _ACC_CAP), 1)
    iota_cls = jax.lax.broadcasted_iota(jnp.int32, (1, 128), 1)
    neg = jnp.float32(_NEG)
    msc_s[...] = msc_ref[...]

    def binit(j, bms):
        base = pl.multiple_of(j * _CHUNK, _CHUNK)
        out = []
        for b in range(nb):
            mj = jnp.max(msc_s[b:b + 1, pl.ds(base, _CHUNK)])
            out.append(jnp.where(iota_blk == j, mj, bms[b]))
        return tuple(out)
    bmax0 = jax.lax.fori_loop(
        0, nblk, binit,
        tuple(jnp.full((1, nblk), _NEG, jnp.float32) for _ in range(nb)))

    def _flag(m, kth, nacc):
        return ((m > neg) & ((nacc < _MAX_INST)
                             | ((m >= kth) & (nacc < _ACC_CAP)))
                ).astype(jnp.int32)

    def _mk_state(bm):
        m0 = jnp.max(bm, keepdims=True)
        kth0 = jnp.full((1, 1), _NEG, jnp.float32)
        nacc0 = jnp.zeros((1, 1), jnp.int32)
        return (
            bm,
            jnp.zeros((1, 128), jnp.int32),
            jnp.full((1, _ACC_CAP), _NEG, jnp.float32),
            jnp.zeros((1, _ACC_CAP), jnp.float32),
            jnp.zeros((1, _ACC_CAP), jnp.float32),
            jnp.zeros((1, _ACC_CAP), jnp.float32),
            jnp.zeros((1, _ACC_CAP), jnp.float32),
            jnp.zeros((1, _ACC_CAP), jnp.float32),
            jnp.full((1, _ACC_CAP), -1, jnp.int32),
            jnp.zeros((1, _ACC_CAP), jnp.int32),
            nacc0,
            m0,
            kth0,
            _flag(m0, kth0, nacc0),
        )

    def _step(b, st):
        (bmax, counts, aS, aY1, aX1, aY2, aX2, aA, aC, aR, nacc,
         m, kth, flag) = st
        live = flag != 0
        j = jnp.min(jnp.where(bmax == m, iota_blk, nblk - 1))
        base = pl.multiple_of(j * _CHUNK, _CHUNK)
        mblk = msc_s[b:b + 1, pl.ds(base, _CHUNK)]
        lane = jnp.min(jnp.where(mblk == m, iota_chunk, _CHUNK - 1),
                       keepdims=True)
        pick = iota_chunk == lane
        ci_ = jnp.sum(jnp.where(
            pick, cid_ref[b:b + 1, pl.ds(base, _CHUNK)], 0), keepdims=True)
        by1 = jnp.sum(jnp.where(
            pick, y1_ref[b:b + 1, pl.ds(base, _CHUNK)], 0.0), keepdims=True)
        bx1 = jnp.sum(jnp.where(
            pick, x1_ref[b:b + 1, pl.ds(base, _CHUNK)], 0.0), keepdims=True)
        by2 = jnp.sum(jnp.where(
            pick, y2_ref[b:b + 1, pl.ds(base, _CHUNK)], 0.0), keepdims=True)
        bx2 = jnp.sum(jnp.where(
            pick, x2_ref[b:b + 1, pl.ds(base, _CHUNK)], 0.0), keepdims=True)
        ba = (by2 - by1) * (bx2 - bx1)

        yy1 = jnp.maximum(by1, aY1)
        xx1 = jnp.maximum(bx1, aX1)
        yy2 = jnp.minimum(by2, aY2)
        xx2 = jnp.minimum(bx2, aX2)
        inter = jnp.maximum(0.0, yy2 - yy1) * jnp.maximum(0.0, xx2 - xx1)
        union = ba + aA - inter
        iou = jnp.where(union > 0, inter / jnp.maximum(union, 1e-12), 0.0)
        samecls = (aC == ci_) & (iota_cap < nacc)
        suppressed = jnp.any(samecls & (iou > _NMS_THR), keepdims=True)
        cnt_c = jnp.sum(jnp.where(iota_cls == ci_, counts, 0), keepdims=True)
        accept = live & jnp.logical_not(suppressed) & (cnt_c < _MAX_INST)

        slotmask = (iota_cap == nacc) & accept
        aS = jnp.where(slotmask, m, aS)
        aY1 = jnp.where(slotmask, by1, aY1)
        aX1 = jnp.where(slotmask, bx1, aX1)
        aY2 = jnp.where(slotmask, by2, aY2)
        aX2 = jnp.where(slotmask, bx2, aX2)
        aA = jnp.where(slotmask, ba, aA)
        aC = jnp.where(slotmask, ci_, aC)
        aR = jnp.where(slotmask, cnt_c, aR)
        counts = jnp.where((iota_cls == ci_) & accept, counts + 1, counts)
        kth = jnp.where(accept & (nacc == _MAX_INST - 1), m, kth)
        nacc = nacc + accept.astype(jnp.int32)
        mblk_new = jnp.where(pick & live, neg, mblk)
        msc_s[b:b + 1, pl.ds(base, _CHUNK)] = mblk_new
        bmax = jnp.where((iota_blk == j) & live,
                         jnp.max(mblk_new, keepdims=True), bmax)
        m = jnp.max(bmax, keepdims=True)
        return (bmax, counts, aS, aY1, aX1, aY2, aX2, aA, aC, aR, nacc,
                m, kth, _flag(m, kth, nacc))

    def cond(sts):
        total = sts[0][13]
        for b in range(1, nb):
            total = total + sts[b][13]
        return jnp.sum(total) > 0

    def body(sts):
        for _ in range(4):
            sts = tuple(_step(b, sts[b]) for b in range(nb))
        return sts

    sts = jax.lax.while_loop(cond, body, tuple(_mk_state(bm) for bm in bmax0))

    rowi = jax.lax.broadcasted_iota(jnp.int32, (_MAX_INST, 6), 0)
    colj = jax.lax.broadcasted_iota(jnp.int32, (_MAX_INST, 6), 1)
    bigi = jnp.int32(1 << 30)
    grs = [sts[b][8] * _MAX_INST + sts[b][9] for b in range(nb)]

    def _order(b, t, st2):
        det, avail_i = st2
        aS, aY1, aX1, aY2, aX2, aA, aC, aR = sts[b][2:10]
        gr = grs[b]
        avail = avail_i != 0
        mm = jnp.max(jnp.where(avail, aS, neg), keepdims=True)
        cand = avail & (aS == mm)
        g = jnp.min(jnp.where(cand, gr, bigi), keepdims=True)
        slot = jnp.min(jnp.where(cand & (gr == g), iota_cap, _ACC_CAP),
                       keepdims=True)
        pickc = iota_cap == slot
        vy1 = jnp.sum(jnp.where(pickc, aY1, 0.0), keepdims=True)
        vx1 = jnp.sum(jnp.where(pickc, aX1, 0.0), keepdims=True)
        vy2 = jnp.sum(jnp.where(pickc, aY2, 0.0), keepdims=True)
        vx2 = jnp.sum(jnp.where(pickc, aX2, 0.0), keepdims=True)
        vc = jnp.sum(jnp.where(pickc, aC, 0), keepdims=True
                     ).astype(jnp.float32)
        vs = jnp.sum(jnp.where(pickc, aS, 0.0), keepdims=True)
        ok = mm > neg
        rowm = (rowi == t) & ok
        rowvals = jnp.where(colj == 0, vy1,
                   jnp.where(colj == 1, vx1,
                    jnp.where(colj == 2, vy2,
                     jnp.where(colj == 3, vx2,
                      jnp.where(colj == 4, vc, vs)))))
        det = jnp.where(rowm, rowvals, det)
        avail_i = jnp.where(iota_cap != slot, avail_i, 0)
        return det, avail_i

    def obody(t, st_all):
        return tuple(_order(b, t, st_all[b]) for b in range(nb))

    det0s = tuple(
        (jnp.zeros((_MAX_INST, 6), jnp.float32),
         (iota_cap < sts[b][10]).astype(jnp.int32))
        for b in range(nb))
    st_all = jax.lax.fori_loop(0, _MAX_INST, obody, det0s)
    for b in range(nb):
        out_ref[b] = st_all[b][0]


def kernel(rois, mrcnn_class, mrcnn_bbox, image_meta):
    B, N, C = mrcnn_class.shape
    shift = jnp.asarray(np.array([0.0, 0.0, 1.0, 1.0], dtype=np.float32))
    image_shape = image_meta[0, 4:7]
    scale = jnp.concatenate([image_shape[:2], image_shape[:2]]) - 1.0
    windows = (image_meta[:, 7:11] - shift) / scale

    probs_t = jnp.transpose(mrcnn_class, (2, 0, 1))
    bbox_t = jnp.transpose(mrcnn_bbox, (0, 2, 3, 1))
    rois_t = jnp.transpose(rois, (0, 2, 1))

    NL = 2048
    T = -(-N // NL)
    n_lanes = T * NL
    n_arr = jnp.full((1, 1), N, jnp.int32)

    outs = pl.pallas_call(
        _prep_kernel,
        grid=(T,),
        in_specs=[
            pl.BlockSpec((1, 1), lambda t: (0, 0)),
            pl.BlockSpec((C, B, NL), lambda t: (0, 0, t)),
            pl.BlockSpec((B, C, 4, NL), lambda t: (0, 0, 0, t)),
            pl.BlockSpec((B, 4, NL), lambda t: (0, 0, t)),
            pl.BlockSpec((B, 4), lambda t: (0, 0)),
        ],
        out_specs=[
            pl.BlockSpec((B, NL), lambda t: (0, t)),
            pl.BlockSpec((B, NL), lambda t: (0, t)),
            pl.BlockSpec((B, NL), lambda t: (0, t)),
            pl.BlockSpec((B, NL), lambda t: (0, t)),
            pl.BlockSpec((B, NL), lambda t: (0, t)),
            pl.BlockSpec((B, NL), lambda t: (0, t)),
        ],
        out_shape=[
            jax.ShapeDtypeStruct((B, n_lanes), jnp.float32),
            jax.ShapeDtypeStruct((B, n_lanes), jnp.float32),
            jax.ShapeDtypeStruct((B, n_lanes), jnp.float32),
            jax.ShapeDtypeStruct((B, n_lanes), jnp.float32),
            jax.ShapeDtypeStruct((B, n_lanes), jnp.float32),
            jax.ShapeDtypeStruct((B, n_lanes), jnp.int32),
        ],
    )(n_arr, probs_t, bbox_t, rois_t, windows)
    y1, x1, y2, x2, msc, cid = outs

    det = pl.pallas_call(
        _select_kernel,
        in_specs=[
            pl.BlockSpec((B, n_lanes), lambda: (0, 0)),
            pl.BlockSpec((B, n_lanes), lambda: (0, 0)),
            pl.BlockSpec((B, n_lanes), lambda: (0, 0)),
            pl.BlockSpec((B, n_lanes), lambda: (0, 0)),
            pl.BlockSpec((B, n_lanes), lambda: (0, 0)),
            pl.BlockSpec((B, n_lanes), lambda: (0, 0)),
        ],
        out_specs=pl.BlockSpec((B, _MAX_INST, 6), lambda: (0, 0, 0)),
        out_shape=jax.ShapeDtypeStruct((B, _MAX_INST, 6), jnp.float32),
        scratch_shapes=[pltpu.VMEM((B, n_lanes), jnp.float32)],
    )(y1, x1, y2, x2, msc, cid)
    return det
